# baseline (speedup 1.0000x reference)
import jax
import jax.numpy as jnp
from jax import lax
from jax.experimental import pallas as pl
from jax.experimental.pallas import tpu as pltpu
from jax.experimental.pallas import tpu_sc as plsc

N = 10000
E = 320000
IN_DIM = 128
OUT_DIM = 64

NC = 2
NS = 16
CH = 80
NCHUNK = 125
E_TILE = CH * NCHUNK
N_SP = 10240
ROWS_PER_TILE = N_SP // NS
ZBLK = 128
DEG_W = 16

_sc_mesh = plsc.VectorSubcoreMesh(
    core_axis_name="c", subcore_axis_name="s", num_cores=NC, num_subcores=NS)


def _sc_agg_body(ei_hbm, y_hbm, agg_out, deg_out,
                 sidx, didx, rows, rows2, ones_v, agg_sp, deg_sp,
                 sem_g, sem_s, sem_d):
  c = lax.axis_index("c")
  s = lax.axis_index("s")

  zero16 = jnp.zeros((16,), jnp.float32)

  def zrow(r, carry):
    for cc in range(OUT_DIM // 16):
      rows[r, pl.ds(cc * 16, 16)] = zero16
    ones_v[r, pl.ds(0, 16)] = zero16
    return carry

  lax.fori_loop(0, ZBLK, zrow, 0)

  row_base = s * ROWS_PER_TILE
  for i in range(ROWS_PER_TILE // ZBLK):
    pltpu.sync_copy(rows.at[pl.ds(0, ZBLK)],
                    agg_sp.at[pl.ds(row_base + i * ZBLK, ZBLK)])
    pltpu.sync_copy(ones_v.at[pl.ds(0, ZBLK)],
                    deg_sp.at[pl.ds(row_base + i * ZBLK, ZBLK)])

  one16 = jnp.ones((16,), jnp.float32)

  def orow(r, carry):
    ones_v[r, pl.ds(0, 16)] = one16
    return carry

  lax.fori_loop(0, CH, orow, 0)

  pltpu.sync_copy(ei_hbm.at[0, c, s], sidx)
  pltpu.sync_copy(ei_hbm.at[1, c, s], didx)

  plsc.subcore_barrier()

  def start_gather(j, buf):
    pltpu.async_copy(y_hbm.at[sidx.at[j]], buf, sem_g)

  def wait_gather(buf):
    pltpu.make_async_copy(y_hbm.at[sidx.at[0]], buf, sem_g).wait()

  def start_scatter(j, buf):
    pltpu.async_copy(buf, agg_sp.at[didx.at[j]], sem_s, add=True)
    pltpu.async_copy(ones_v, deg_sp.at[didx.at[j]], sem_d, add=True)

  def wait_scatter():
    pltpu.make_async_copy(rows, agg_sp.at[didx.at[0]], sem_s).wait()
    pltpu.make_async_copy(ones_v, deg_sp.at[didx.at[0]], sem_d).wait()

  start_gather(0, rows)

  def chunk2(jj, carry):
    j0 = jj * 2
    wait_gather(rows)
    jax.lax.cond(jj > 0, wait_scatter, lambda: None)
    start_gather(j0 + 1, rows2)
    start_scatter(j0, rows)
    wait_gather(rows2)
    wait_scatter()
    jax.lax.cond(jj < NCHUNK // 2 - 1,
                 lambda: start_gather(j0 + 2, rows), lambda: None)
    start_scatter(j0 + 1, rows2)
    return carry

  lax.fori_loop(0, NCHUNK // 2, chunk2, 0)
  wait_scatter()

  plsc.subcore_barrier()

  pltpu.sync_copy(agg_sp.at[pl.ds(row_base, ROWS_PER_TILE)],
                  agg_out.at[c, pl.ds(row_base, ROWS_PER_TILE)])
  pltpu.sync_copy(deg_sp.at[pl.ds(row_base, ROWS_PER_TILE)],
                  deg_out.at[c, pl.ds(row_base, ROWS_PER_TILE)])


_sc_agg = pl.kernel(
    _sc_agg_body,
    out_type=(jax.ShapeDtypeStruct((NC, N_SP, OUT_DIM), jnp.float32),
              jax.ShapeDtypeStruct((NC, N_SP, DEG_W), jnp.float32)),
    mesh=_sc_mesh,
    scratch_types=[
        pltpu.VMEM((NCHUNK, CH), jnp.int32),
        pltpu.VMEM((NCHUNK, CH), jnp.int32),
        pltpu.VMEM((CH, OUT_DIM), jnp.float32),
        pltpu.VMEM((CH, OUT_DIM), jnp.float32),
        pltpu.VMEM((CH, DEG_W), jnp.float32),
        pltpu.VMEM_SHARED((N_SP, OUT_DIM), jnp.float32),
        pltpu.VMEM_SHARED((N_SP, DEG_W), jnp.float32),
        pltpu.SemaphoreType.DMA,
        pltpu.SemaphoreType.DMA,
        pltpu.SemaphoreType.DMA,
    ],
    compiler_params=pltpu.CompilerParams(use_tc_tiling_on_sc=False),
)


def _mm_body(x_ref, wt_ref, o_ref):
  o_ref[...] = jnp.dot(x_ref[...], wt_ref[...],
                       preferred_element_type=jnp.float32)


def _finalize_body(agg_ref, deg_ref, b_ref, o_ref):
  ssum = agg_ref[0] + agg_ref[1]
  d = deg_ref[0, :, 0:1] + deg_ref[1, :, 0:1]
  d = jnp.maximum(d, 1.0)
  o_ref[...] = jnp.maximum(ssum / d + b_ref[...], 0.0)


_MM_BLK = 1000


def _matmul(x, wt):
  return pl.pallas_call(
      _mm_body,
      grid=(N // _MM_BLK,),
      in_specs=[
          pl.BlockSpec((_MM_BLK, IN_DIM), lambda i: (i, 0)),
          pl.BlockSpec((IN_DIM, OUT_DIM), lambda i: (0, 0)),
      ],
      out_specs=pl.BlockSpec((_MM_BLK, OUT_DIM), lambda i: (i, 0)),
      out_shape=jax.ShapeDtypeStruct((N, OUT_DIM), jnp.float32),
  )(x, wt)


def _finalize(agg2, deg2, b2):
  return pl.pallas_call(
      _finalize_body,
      grid=(N // _MM_BLK,),
      in_specs=[
          pl.BlockSpec((NC, _MM_BLK, OUT_DIM), lambda i: (0, i, 0)),
          pl.BlockSpec((NC, _MM_BLK, DEG_W), lambda i: (0, i, 0)),
          pl.BlockSpec((1, OUT_DIM), lambda i: (0, 0)),
      ],
      out_specs=pl.BlockSpec((_MM_BLK, OUT_DIM), lambda i: (i, 0)),
      out_shape=jax.ShapeDtypeStruct((N, OUT_DIM), jnp.float32),
  )(agg2, deg2, b2)


def kernel(node_features, edge_index, W, b):
  ei = edge_index.astype(jnp.int32).reshape(2, NC, NS, NCHUNK, CH)
  y = _matmul(node_features, W.T)
  agg2, deg2 = _sc_agg(ei, y)
  return _finalize(agg2, deg2, b.reshape(1, OUT_DIM))


# speedup vs baseline: 1.1522x; 1.1522x over previous
import jax
import jax.numpy as jnp
from jax import lax
from jax.experimental import pallas as pl
from jax.experimental.pallas import tpu as pltpu
from jax.experimental.pallas import tpu_sc as plsc

N = 10000
E = 320000
IN_DIM = 128
OUT_DIM = 64

NC = 2
NS = 16
CH = 128
NCHUNKS_TOT = E // CH
NCHUNK = NCHUNKS_TOT // (NC * NS)
NEXTRA = NCHUNKS_TOT - NCHUNK * NC * NS
N_SP = 10240
ROWS_PER_TILE = N_SP // NS
ZBLK = 128
DEG_W = 16

_sc_mesh = plsc.VectorSubcoreMesh(
    core_axis_name="c", subcore_axis_name="s", num_cores=NC, num_subcores=NS)


def _sc_agg_body(ei_hbm, y_hbm, agg_out, deg_out,
                 sidx, didx, rows, rows2, ones_v, agg_sp, deg_sp,
                 sem_g, sem_s, sem_d):
  c = lax.axis_index("c")
  s = lax.axis_index("s")

  zero16 = jnp.zeros((16,), jnp.float32)

  def zrow(r, carry):
    for cc in range(OUT_DIM // 16):
      rows[r, pl.ds(cc * 16, 16)] = zero16
    ones_v[r, pl.ds(0, 16)] = zero16
    return carry

  lax.fori_loop(0, ZBLK, zrow, 0)

  row_base = s * ROWS_PER_TILE
  for i in range(ROWS_PER_TILE // ZBLK):
    pltpu.sync_copy(rows.at[pl.ds(0, ZBLK)],
                    agg_sp.at[pl.ds(row_base + i * ZBLK, ZBLK)])
    pltpu.sync_copy(ones_v.at[pl.ds(0, ZBLK)],
                    deg_sp.at[pl.ds(row_base + i * ZBLK, ZBLK)])

  one16 = jnp.ones((16,), jnp.float32)

  def orow(r, carry):
    ones_v[r, pl.ds(0, 16)] = one16
    return carry

  lax.fori_loop(0, CH, orow, 0)

  w = c * NS + s
  pltpu.sync_copy(ei_hbm.at[0, pl.ds(w * NCHUNK, NCHUNK)],
                  sidx.at[pl.ds(0, NCHUNK)])
  pltpu.sync_copy(ei_hbm.at[1, pl.ds(w * NCHUNK, NCHUNK)],
                  didx.at[pl.ds(0, NCHUNK)])

  @pl.when(w < NEXTRA)
  def _stage_extra():
    pltpu.sync_copy(ei_hbm.at[0, pl.ds(NCHUNK * NC * NS + w, 1)],
                    sidx.at[pl.ds(NCHUNK, 1)])
    pltpu.sync_copy(ei_hbm.at[1, pl.ds(NCHUNK * NC * NS + w, 1)],
                    didx.at[pl.ds(NCHUNK, 1)])

  plsc.subcore_barrier()

  def start_gather(j, buf):
    pltpu.async_copy(y_hbm.at[sidx.at[j]], buf, sem_g)

  def wait_gather(buf):
    pltpu.make_async_copy(y_hbm.at[sidx.at[0]], buf, sem_g).wait()

  def start_scatter(j, buf):
    pltpu.async_copy(buf, agg_sp.at[didx.at[j]], sem_s, add=True)
    pltpu.async_copy(ones_v, deg_sp.at[didx.at[j]], sem_d, add=True)

  def wait_scatter():
    pltpu.make_async_copy(rows, agg_sp.at[didx.at[0]], sem_s).wait()
    pltpu.make_async_copy(ones_v, deg_sp.at[didx.at[0]], sem_d).wait()

  start_gather(0, rows)

  def chunk2(jj, carry):
    j0 = jj * 2
    wait_gather(rows)
    jax.lax.cond(jj > 0, wait_scatter, lambda: None)
    start_gather(j0 + 1, rows2)
    start_scatter(j0, rows)
    wait_gather(rows2)
    wait_scatter()
    jax.lax.cond(jj < NCHUNK // 2 - 1,
                 lambda: start_gather(j0 + 2, rows), lambda: None)
    start_scatter(j0 + 1, rows2)
    return carry

  lax.fori_loop(0, NCHUNK // 2, chunk2, 0)

  def tail():
    start_gather(NCHUNK, rows)
    wait_scatter()
    wait_gather(rows)
    start_scatter(NCHUNK, rows)
    wait_scatter()

  jax.lax.cond(w < NEXTRA, tail, wait_scatter)

  plsc.subcore_barrier()

  pltpu.sync_copy(agg_sp.at[pl.ds(row_base, ROWS_PER_TILE)],
                  agg_out.at[c, pl.ds(row_base, ROWS_PER_TILE)])
  pltpu.sync_copy(deg_sp.at[pl.ds(row_base, ROWS_PER_TILE)],
                  deg_out.at[c, pl.ds(row_base, ROWS_PER_TILE)])


_sc_agg = pl.kernel(
    _sc_agg_body,
    out_type=(jax.ShapeDtypeStruct((NC, N_SP, OUT_DIM), jnp.float32),
              jax.ShapeDtypeStruct((NC, N_SP, DEG_W), jnp.float32)),
    mesh=_sc_mesh,
    scratch_types=[
        pltpu.VMEM((NCHUNK + 1, CH), jnp.int32),
        pltpu.VMEM((NCHUNK + 1, CH), jnp.int32),
        pltpu.VMEM((CH, OUT_DIM), jnp.float32),
        pltpu.VMEM((CH, OUT_DIM), jnp.float32),
        pltpu.VMEM((CH, DEG_W), jnp.float32),
        pltpu.VMEM_SHARED((N_SP, OUT_DIM), jnp.float32),
        pltpu.VMEM_SHARED((N_SP, DEG_W), jnp.float32),
        pltpu.SemaphoreType.DMA,
        pltpu.SemaphoreType.DMA,
        pltpu.SemaphoreType.DMA,
    ],
    compiler_params=pltpu.CompilerParams(use_tc_tiling_on_sc=False),
)


def _mm_body(x_ref, wt_ref, o_ref):
  o_ref[...] = jnp.dot(x_ref[...], wt_ref[...],
                       preferred_element_type=jnp.float32)


def _finalize_body(agg_ref, deg_ref, b_ref, o_ref):
  ssum = agg_ref[0] + agg_ref[1]
  d = deg_ref[0, :, 0:1] + deg_ref[1, :, 0:1]
  d = jnp.maximum(d, 1.0)
  o_ref[...] = jnp.maximum(ssum / d + b_ref[...], 0.0)


_MM_BLK = 1000


def _matmul(x, wt):
  return pl.pallas_call(
      _mm_body,
      grid=(N // _MM_BLK,),
      in_specs=[
          pl.BlockSpec((_MM_BLK, IN_DIM), lambda i: (i, 0)),
          pl.BlockSpec((IN_DIM, OUT_DIM), lambda i: (0, 0)),
      ],
      out_specs=pl.BlockSpec((_MM_BLK, OUT_DIM), lambda i: (i, 0)),
      out_shape=jax.ShapeDtypeStruct((N, OUT_DIM), jnp.float32),
  )(x, wt)


def _finalize(agg2, deg2, b2):
  return pl.pallas_call(
      _finalize_body,
      grid=(N // _MM_BLK,),
      in_specs=[
          pl.BlockSpec((NC, _MM_BLK, OUT_DIM), lambda i: (0, i, 0)),
          pl.BlockSpec((NC, _MM_BLK, DEG_W), lambda i: (0, i, 0)),
          pl.BlockSpec((1, OUT_DIM), lambda i: (0, 0)),
      ],
      out_specs=pl.BlockSpec((_MM_BLK, OUT_DIM), lambda i: (i, 0)),
      out_shape=jax.ShapeDtypeStruct((N, OUT_DIM), jnp.float32),
  )(agg2, deg2, b2)


def kernel(node_features, edge_index, W, b):
  ei = edge_index.astype(jnp.int32).reshape(2, NCHUNKS_TOT, CH)
  y = _matmul(node_features, W.T)
  agg2, deg2 = _sc_agg(ei, y)
  return _finalize(agg2, deg2, b.reshape(1, OUT_DIM))


# speedup vs baseline: 1.4227x; 1.2347x over previous
import jax
import jax.numpy as jnp
from jax import lax
from jax.experimental import pallas as pl
from jax.experimental.pallas import tpu as pltpu
from jax.experimental.pallas import tpu_sc as plsc

N = 10000
E = 320000
IN_DIM = 128
OUT_DIM = 64

NC = 2
NS = 16
CH = 128
NCHUNKS_TOT = E // CH
NCHUNK = NCHUNKS_TOT // (NC * NS)
NEXTRA = NCHUNKS_TOT - NCHUNK * NC * NS
NBUF = 6
N_SP = 10240
ROWS_PER_TILE = N_SP // NS
ZBLK = 128
DEG_W = 16

_sc_mesh = plsc.VectorSubcoreMesh(
    core_axis_name="c", subcore_axis_name="s", num_cores=NC, num_subcores=NS)


def _sc_agg_body(ei_hbm, y_hbm, agg_out, deg_out,
                 sidx, didx, rows0, rows1, rows2, rows3, rows4, rows5,
                 ones_v, agg_sp, deg_sp,
                 sg0, sg1, sg2, sg3, sg4, sg5,
                 ss0, ss1, ss2, ss3, ss4, ss5, sem_d):
  c = lax.axis_index("c")
  s = lax.axis_index("s")

  zero16 = jnp.zeros((16,), jnp.float32)

  def zrow(r, carry):
    for cc in range(OUT_DIM // 16):
      rows0[r, pl.ds(cc * 16, 16)] = zero16
    ones_v[r, pl.ds(0, 16)] = zero16
    return carry

  lax.fori_loop(0, ZBLK, zrow, 0)

  row_base = s * ROWS_PER_TILE
  for i in range(ROWS_PER_TILE // ZBLK):
    pltpu.sync_copy(rows0.at[pl.ds(0, ZBLK)],
                    agg_sp.at[pl.ds(row_base + i * ZBLK, ZBLK)])
    pltpu.sync_copy(ones_v.at[pl.ds(0, ZBLK)],
                    deg_sp.at[pl.ds(row_base + i * ZBLK, ZBLK)])

  one16 = jnp.ones((16,), jnp.float32)

  def orow(r, carry):
    ones_v[r, pl.ds(0, 16)] = one16
    return carry

  lax.fori_loop(0, CH, orow, 0)

  w = c * NS + s
  pltpu.sync_copy(ei_hbm.at[0, pl.ds(w * NCHUNK, NCHUNK)],
                  sidx.at[pl.ds(0, NCHUNK)])
  pltpu.sync_copy(ei_hbm.at[1, pl.ds(w * NCHUNK, NCHUNK)],
                  didx.at[pl.ds(0, NCHUNK)])

  @pl.when(w < NEXTRA)
  def _stage_extra():
    pltpu.sync_copy(ei_hbm.at[0, pl.ds(NCHUNK * NC * NS + w, 1)],
                    sidx.at[pl.ds(NCHUNK, 1)])
    pltpu.sync_copy(ei_hbm.at[1, pl.ds(NCHUNK * NC * NS + w, 1)],
                    didx.at[pl.ds(NCHUNK, 1)])

  plsc.subcore_barrier()

  def start_gather(j, buf, sem):
    pltpu.async_copy(y_hbm.at[sidx.at[j]], buf, sem)

  def wait_gather(buf, sem):
    pltpu.make_async_copy(y_hbm.at[sidx.at[0]], buf, sem).wait()

  def start_scatter(j, buf, sem):
    pltpu.async_copy(buf, agg_sp.at[didx.at[j]], sem, add=True)
    pltpu.async_copy(ones_v, deg_sp.at[didx.at[j]], sem_d, add=True)

  def wait_scatter(buf, sem):
    pltpu.make_async_copy(buf, agg_sp.at[didx.at[0]], sem).wait()

  def wait_deg():
    pltpu.make_async_copy(ones_v, deg_sp.at[didx.at[0]], sem_d).wait()

  bufs = (rows0, rows1, rows2, rows3, rows4, rows5)
  gsems = (sg0, sg1, sg2, sg3, sg4, sg5)
  ssems = (ss0, ss1, ss2, ss3, ss4, ss5)

  for b in range(NBUF):
    start_gather(b, bufs[b], gsems[b])

  def round6(k, carry):
    j0 = k * NBUF
    for b in range(NBUF):
      wait_gather(bufs[b], gsems[b])
      start_scatter(j0 + b, bufs[b], ssems[b])

      @pl.when(k < NCHUNK // NBUF - 1)
      def _next():
        wait_scatter(bufs[b], ssems[b])
        start_gather(j0 + NBUF + b, bufs[b], gsems[b])

    return carry

  lax.fori_loop(0, NCHUNK // NBUF, round6, 0)
  for b in range(NBUF):
    wait_scatter(bufs[b], ssems[b])

  @pl.when(w < NEXTRA)
  def _tail():
    start_gather(NCHUNK, rows0, sg0)
    wait_gather(rows0, sg0)
    start_scatter(NCHUNK, rows0, ss0)
    wait_scatter(rows0, ss0)

  def drain_deg(j, carry):
    wait_deg()
    return carry

  lax.fori_loop(0, NCHUNK, drain_deg, 0)

  @pl.when(w < NEXTRA)
  def _drain_tail():
    wait_deg()

  plsc.subcore_barrier()

  pltpu.sync_copy(agg_sp.at[pl.ds(row_base, ROWS_PER_TILE)],
                  agg_out.at[c, pl.ds(row_base, ROWS_PER_TILE)])
  pltpu.sync_copy(deg_sp.at[pl.ds(row_base, ROWS_PER_TILE)],
                  deg_out.at[c, pl.ds(row_base, ROWS_PER_TILE)])


_sc_agg = pl.kernel(
    _sc_agg_body,
    out_type=(jax.ShapeDtypeStruct((NC, N_SP, OUT_DIM), jnp.float32),
              jax.ShapeDtypeStruct((NC, N_SP, DEG_W), jnp.float32)),
    mesh=_sc_mesh,
    scratch_types=[
        pltpu.VMEM((NCHUNK + 1, CH), jnp.int32),
        pltpu.VMEM((NCHUNK + 1, CH), jnp.int32),
        pltpu.VMEM((CH, OUT_DIM), jnp.float32),
        pltpu.VMEM((CH, OUT_DIM), jnp.float32),
        pltpu.VMEM((CH, OUT_DIM), jnp.float32),
        pltpu.VMEM((CH, OUT_DIM), jnp.float32),
        pltpu.VMEM((CH, OUT_DIM), jnp.float32),
        pltpu.VMEM((CH, OUT_DIM), jnp.float32),
        pltpu.VMEM((CH, DEG_W), jnp.float32),
        pltpu.VMEM_SHARED((N_SP, OUT_DIM), jnp.float32),
        pltpu.VMEM_SHARED((N_SP, DEG_W), jnp.float32),
    ] + [pltpu.SemaphoreType.DMA] * 13,
    compiler_params=pltpu.CompilerParams(use_tc_tiling_on_sc=False),
)


def _mm_body(x_ref, wt_ref, o_ref):
  o_ref[...] = jnp.dot(x_ref[...], wt_ref[...],
                       preferred_element_type=jnp.float32)


def _finalize_body(agg_ref, deg_ref, b_ref, o_ref):
  ssum = agg_ref[0] + agg_ref[1]
  d = deg_ref[0, :, 0:1] + deg_ref[1, :, 0:1]
  d = jnp.maximum(d, 1.0)
  o_ref[...] = jnp.maximum(ssum / d + b_ref[...], 0.0)


_MM_BLK = 1000


def _matmul(x, wt):
  return pl.pallas_call(
      _mm_body,
      grid=(N // _MM_BLK,),
      in_specs=[
          pl.BlockSpec((_MM_BLK, IN_DIM), lambda i: (i, 0)),
          pl.BlockSpec((IN_DIM, OUT_DIM), lambda i: (0, 0)),
      ],
      out_specs=pl.BlockSpec((_MM_BLK, OUT_DIM), lambda i: (i, 0)),
      out_shape=jax.ShapeDtypeStruct((N, OUT_DIM), jnp.float32),
  )(x, wt)


def _finalize(agg2, deg2, b2):
  return pl.pallas_call(
      _finalize_body,
      grid=(N // _MM_BLK,),
      in_specs=[
          pl.BlockSpec((NC, _MM_BLK, OUT_DIM), lambda i: (0, i, 0)),
          pl.BlockSpec((NC, _MM_BLK, DEG_W), lambda i: (0, i, 0)),
          pl.BlockSpec((1, OUT_DIM), lambda i: (0, 0)),
      ],
      out_specs=pl.BlockSpec((_MM_BLK, OUT_DIM), lambda i: (i, 0)),
      out_shape=jax.ShapeDtypeStruct((N, OUT_DIM), jnp.float32),
  )(agg2, deg2, b2)


def kernel(node_features, edge_index, W, b):
  ei = edge_index.astype(jnp.int32).reshape(2, NCHUNKS_TOT, CH)
  y = _matmul(node_features, W.T)
  agg2, deg2 = _sc_agg(ei, y)
  return _finalize(agg2, deg2, b.reshape(1, OUT_DIM))


# speedup vs baseline: 1.4642x; 1.0292x over previous
import jax
import jax.numpy as jnp
from jax import lax
from jax.experimental import pallas as pl
from jax.experimental.pallas import tpu as pltpu
from jax.experimental.pallas import tpu_sc as plsc

N = 10000
E = 320000
IN_DIM = 128
OUT_DIM = 64

NC = 2
NS = 16
CH = 128
NCHUNKS_TOT = E // CH
NCHUNK = NCHUNKS_TOT // (NC * NS)
NEXTRA = NCHUNKS_TOT - NCHUNK * NC * NS
NBUF = 6
N_SP = 10240
ROWS_PER_TILE = N_SP // NS
ZBLK = 128
DEG_W = 16

_sc_mesh = plsc.VectorSubcoreMesh(
    core_axis_name="c", subcore_axis_name="s", num_cores=NC, num_subcores=NS)


def _sc_agg_body(ei_hbm, y_hbm, agg_out, deg_out,
                 sidx, didx, rows0, rows1, rows2, rows3, rows4, rows5,
                 ones_v, agg_sp, deg_sp,
                 sg0, sg1, sg2, sg3, sg4, sg5,
                 ss0, ss1, ss2, ss3, ss4, ss5, sem_d):
  c = lax.axis_index("c")
  s = lax.axis_index("s")

  zero16 = jnp.zeros((16,), jnp.float32)

  def zrow(r, carry):
    for cc in range(OUT_DIM // 16):
      rows0[r, pl.ds(cc * 16, 16)] = zero16
    ones_v[r, pl.ds(0, 16)] = zero16
    return carry

  lax.fori_loop(0, ZBLK, zrow, 0)

  row_base = s * ROWS_PER_TILE
  for i in range(ROWS_PER_TILE // ZBLK):
    pltpu.sync_copy(rows0.at[pl.ds(0, ZBLK)],
                    agg_sp.at[pl.ds(row_base + i * ZBLK, ZBLK)])
    pltpu.sync_copy(ones_v.at[pl.ds(0, ZBLK)],
                    deg_sp.at[pl.ds(row_base + i * ZBLK, ZBLK)])

  one16 = jnp.ones((16,), jnp.float32)

  def orow(r, carry):
    ones_v[r, pl.ds(0, 16)] = one16
    return carry

  lax.fori_loop(0, CH, orow, 0)

  w = c * NS + s
  pltpu.sync_copy(ei_hbm.at[0, pl.ds(w * NCHUNK, NCHUNK)],
                  sidx.at[pl.ds(0, NCHUNK)])
  pltpu.sync_copy(ei_hbm.at[1, pl.ds(w * NCHUNK, NCHUNK)],
                  didx.at[pl.ds(0, NCHUNK)])

  @pl.when(w < NEXTRA)
  def _stage_extra():
    pltpu.sync_copy(ei_hbm.at[0, pl.ds(NCHUNK * NC * NS + w, 1)],
                    sidx.at[pl.ds(NCHUNK, 1)])
    pltpu.sync_copy(ei_hbm.at[1, pl.ds(NCHUNK * NC * NS + w, 1)],
                    didx.at[pl.ds(NCHUNK, 1)])

  plsc.subcore_barrier()

  def start_gather(j, buf, sem):
    pltpu.async_copy(y_hbm.at[sidx.at[j]], buf, sem)

  def wait_gather(buf, sem):
    pltpu.make_async_copy(y_hbm.at[sidx.at[0]], buf, sem).wait()

  def start_scatter(j, buf, sem):
    pltpu.async_copy(buf, agg_sp.at[didx.at[j]], sem, add=True)
    pltpu.async_copy(ones_v, deg_sp.at[didx.at[j]], sem_d, add=True)

  def wait_scatter(buf, sem):
    pltpu.make_async_copy(buf, agg_sp.at[didx.at[0]], sem).wait()

  def wait_deg():
    pltpu.make_async_copy(ones_v, deg_sp.at[didx.at[0]], sem_d).wait()

  bufs = (rows0, rows1, rows2, rows3, rows4, rows5)
  gsems = (sg0, sg1, sg2, sg3, sg4, sg5)
  ssems = (ss0, ss1, ss2, ss3, ss4, ss5)

  for b in range(NBUF):
    start_gather(b, bufs[b], gsems[b])

  def round6(k, carry):
    j0 = k * NBUF
    for b in range(NBUF):
      wait_gather(bufs[b], gsems[b])
      start_scatter(j0 + b, bufs[b], ssems[b])

      @pl.when(k < NCHUNK // NBUF - 1)
      def _next():
        wait_scatter(bufs[b], ssems[b])
        start_gather(j0 + NBUF + b, bufs[b], gsems[b])

    return carry

  lax.fori_loop(0, NCHUNK // NBUF, round6, 0)
  for b in range(NBUF):
    wait_scatter(bufs[b], ssems[b])

  @pl.when(w < NEXTRA)
  def _tail():
    start_gather(NCHUNK, rows0, sg0)
    wait_gather(rows0, sg0)
    start_scatter(NCHUNK, rows0, ss0)
    wait_scatter(rows0, ss0)

  def drain_deg(j, carry):
    wait_deg()
    return carry

  lax.fori_loop(0, NCHUNK, drain_deg, 0)

  @pl.when(w < NEXTRA)
  def _drain_tail():
    wait_deg()

  plsc.subcore_barrier()

  pltpu.sync_copy(agg_sp.at[pl.ds(row_base, ROWS_PER_TILE)],
                  agg_out.at[c, pl.ds(row_base, ROWS_PER_TILE)])
  pltpu.sync_copy(deg_sp.at[pl.ds(row_base, ROWS_PER_TILE)],
                  deg_out.at[c, pl.ds(row_base, ROWS_PER_TILE)])


_sc_agg = pl.kernel(
    _sc_agg_body,
    out_type=(jax.ShapeDtypeStruct((NC, N_SP, OUT_DIM), jnp.float32),
              jax.ShapeDtypeStruct((NC, N_SP, DEG_W), jnp.float32)),
    mesh=_sc_mesh,
    scratch_types=[
        pltpu.VMEM((NCHUNK + 1, CH), jnp.int32),
        pltpu.VMEM((NCHUNK + 1, CH), jnp.int32),
        pltpu.VMEM((CH, OUT_DIM), jnp.float32),
        pltpu.VMEM((CH, OUT_DIM), jnp.float32),
        pltpu.VMEM((CH, OUT_DIM), jnp.float32),
        pltpu.VMEM((CH, OUT_DIM), jnp.float32),
        pltpu.VMEM((CH, OUT_DIM), jnp.float32),
        pltpu.VMEM((CH, OUT_DIM), jnp.float32),
        pltpu.VMEM((CH, DEG_W), jnp.float32),
        pltpu.VMEM_SHARED((N_SP, OUT_DIM), jnp.float32),
        pltpu.VMEM_SHARED((N_SP, DEG_W), jnp.float32),
    ] + [pltpu.SemaphoreType.DMA] * 13,
    compiler_params=pltpu.CompilerParams(use_tc_tiling_on_sc=False),
)


def _mm_body(x_ref, wt_ref, o_ref):
  o_ref[...] = jnp.dot(x_ref[...], wt_ref[...],
                       preferred_element_type=jnp.float32)


_MM_BLK = 1000


def _matmul(x, wt):
  return pl.pallas_call(
      _mm_body,
      grid=(N // _MM_BLK,),
      in_specs=[
          pl.BlockSpec((_MM_BLK, IN_DIM), lambda i: (i, 0)),
          pl.BlockSpec((IN_DIM, OUT_DIM), lambda i: (0, 0)),
      ],
      out_specs=pl.BlockSpec((_MM_BLK, OUT_DIM), lambda i: (i, 0)),
      out_shape=jax.ShapeDtypeStruct((N, OUT_DIM), jnp.float32),
  )(x, wt)


FIN_BLK = 80


def _sc_fin_body(agg_hbm, deg_hbm, b_hbm, out_hbm,
                 fa, fb, fd, fd2, fo, bv, sem):
  c = lax.axis_index("c")
  s = lax.axis_index("s")
  w = c * NS + s
  rb = w * (FIN_BLK * 4)
  nb = jnp.where(w == NC * NS - 1, 1, 4)

  pltpu.sync_copy(b_hbm, bv)
  bvs = [bv[pl.ds(16 * cc, 16)] for cc in range(OUT_DIM // 16)]
  onef = jnp.full((16,), 1.0, jnp.float32)

  def blk(i, carry):
    off = rb + i * FIN_BLK
    pltpu.sync_copy(agg_hbm.at[0, pl.ds(off, FIN_BLK)], fa)
    pltpu.sync_copy(agg_hbm.at[1, pl.ds(off, FIN_BLK)], fb)
    pltpu.sync_copy(deg_hbm.at[0, pl.ds(off, FIN_BLK)], fd)
    pltpu.sync_copy(deg_hbm.at[1, pl.ds(off, FIN_BLK)], fd2)

    def row(r, carry2):
      dv = fd[r, pl.ds(0, 16)] + fd2[r, pl.ds(0, 16)]
      rec = onef / jnp.maximum(dv, onef)
      for cc in range(OUT_DIM // 16):
        a = fa[r, pl.ds(16 * cc, 16)] + fb[r, pl.ds(16 * cc, 16)]
        fo[r, pl.ds(16 * cc, 16)] = jnp.maximum(a * rec + bvs[cc], 0.0)
      return carry2

    lax.fori_loop(0, FIN_BLK, row, 0)
    pltpu.sync_copy(fo, out_hbm.at[pl.ds(off, FIN_BLK)])
    return carry

  lax.fori_loop(0, nb, blk, 0)


_sc_fin = pl.kernel(
    _sc_fin_body,
    out_type=jax.ShapeDtypeStruct((N, OUT_DIM), jnp.float32),
    mesh=_sc_mesh,
    scratch_types=[
        pltpu.VMEM((FIN_BLK, OUT_DIM), jnp.float32),
        pltpu.VMEM((FIN_BLK, OUT_DIM), jnp.float32),
        pltpu.VMEM((FIN_BLK, DEG_W), jnp.float32),
        pltpu.VMEM((FIN_BLK, DEG_W), jnp.float32),
        pltpu.VMEM((FIN_BLK, OUT_DIM), jnp.float32),
        pltpu.VMEM((OUT_DIM,), jnp.float32),
        pltpu.SemaphoreType.DMA,
    ],
    compiler_params=pltpu.CompilerParams(use_tc_tiling_on_sc=False),
)


def kernel(node_features, edge_index, W, b):
  ei = edge_index.astype(jnp.int32).reshape(2, NCHUNKS_TOT, CH)
  y = _matmul(node_features, W.T)
  agg2, deg2 = _sc_agg(ei, y)
  return _sc_fin(agg2, deg2, b)


# speedup vs baseline: 1.6293x; 1.1127x over previous
import jax
import jax.numpy as jnp
from jax import lax
from jax.experimental import pallas as pl
from jax.experimental.pallas import tpu as pltpu
from jax.experimental.pallas import tpu_sc as plsc

N = 10000
E = 320000
IN_DIM = 128
OUT_DIM = 64

NC = 2
NS = 16
CH = 128
NCHUNKS_TOT = E // CH
NCHUNK = NCHUNKS_TOT // (NC * NS)
NEXTRA = NCHUNKS_TOT - NCHUNK * NC * NS
NBUF = 6
N_SP = 10240
ROWS_PER_TILE = N_SP // NS
ZBLK = 128
DEG_W = 16

_sc_mesh = plsc.VectorSubcoreMesh(
    core_axis_name="c", subcore_axis_name="s", num_cores=NC, num_subcores=NS)


def _sc_agg_body(ei_hbm, y_hbm, agg_out, deg_out,
                 sidx, didx, rows0, rows1, rows2, rows3, rows4, rows5,
                 ones_v, agg_sp, deg_sp,
                 sg0, sg1, sg2, sg3, sg4, sg5,
                 ss0, ss1, ss2, ss3, ss4, ss5, sem_d):
  c = lax.axis_index("c")
  s = lax.axis_index("s")

  zero16 = jnp.zeros((16,), jnp.float32)

  def zrow(r, carry):
    for cc in range(OUT_DIM // 16):
      rows0[r, pl.ds(cc * 16, 16)] = zero16
    ones_v[r, pl.ds(0, 16)] = zero16
    return carry

  lax.fori_loop(0, ZBLK, zrow, 0)

  row_base = s * ROWS_PER_TILE
  for i in range(ROWS_PER_TILE // ZBLK):
    pltpu.sync_copy(rows0.at[pl.ds(0, ZBLK)],
                    agg_sp.at[pl.ds(row_base + i * ZBLK, ZBLK)])
    pltpu.sync_copy(ones_v.at[pl.ds(0, ZBLK)],
                    deg_sp.at[pl.ds(row_base + i * ZBLK, ZBLK)])

  one16 = jnp.ones((16,), jnp.float32)

  def orow(r, carry):
    ones_v[r, pl.ds(0, 16)] = one16
    return carry

  lax.fori_loop(0, CH, orow, 0)

  w = c * NS + s
  pltpu.sync_copy(ei_hbm.at[0, pl.ds(w * NCHUNK, NCHUNK)],
                  sidx.at[pl.ds(0, NCHUNK)])
  pltpu.sync_copy(ei_hbm.at[1, pl.ds(w * NCHUNK, NCHUNK)],
                  didx.at[pl.ds(0, NCHUNK)])

  @pl.when(w < NEXTRA)
  def _stage_extra():
    pltpu.sync_copy(ei_hbm.at[0, pl.ds(NCHUNK * NC * NS + w, 1)],
                    sidx.at[pl.ds(NCHUNK, 1)])
    pltpu.sync_copy(ei_hbm.at[1, pl.ds(NCHUNK * NC * NS + w, 1)],
                    didx.at[pl.ds(NCHUNK, 1)])

  plsc.subcore_barrier()

  def start_gather(j, buf, sem):
    pltpu.async_copy(y_hbm.at[sidx.at[j]], buf, sem)

  def wait_gather(buf, sem):
    pltpu.make_async_copy(y_hbm.at[sidx.at[0]], buf, sem).wait()

  def start_scatter(j, buf, sem):
    pltpu.async_copy(buf, agg_sp.at[didx.at[j]], sem, add=True)
    pltpu.async_copy(ones_v, deg_sp.at[didx.at[j]], sem_d, add=True)

  def wait_scatter(buf, sem):
    pltpu.make_async_copy(buf, agg_sp.at[didx.at[0]], sem).wait()

  def wait_deg():
    pltpu.make_async_copy(ones_v, deg_sp.at[didx.at[0]], sem_d).wait()

  bufs = (rows0, rows1, rows2, rows3, rows4, rows5)
  gsems = (sg0, sg1, sg2, sg3, sg4, sg5)
  ssems = (ss0, ss1, ss2, ss3, ss4, ss5)

  for b in range(NBUF):
    start_gather(b, bufs[b], gsems[b])

  def round6(k, carry):
    j0 = k * NBUF
    for b in range(NBUF):
      wait_gather(bufs[b], gsems[b])
      start_scatter(j0 + b, bufs[b], ssems[b])

      @pl.when(k < NCHUNK // NBUF - 1)
      def _next():
        wait_scatter(bufs[b], ssems[b])
        start_gather(j0 + NBUF + b, bufs[b], gsems[b])

    return carry

  lax.fori_loop(0, NCHUNK // NBUF, round6, 0)
  for b in range(NBUF):
    wait_scatter(bufs[b], ssems[b])

  @pl.when(w < NEXTRA)
  def _tail():
    start_gather(NCHUNK, rows0, sg0)
    wait_gather(rows0, sg0)
    start_scatter(NCHUNK, rows0, ss0)
    wait_scatter(rows0, ss0)

  def drain_deg(j, carry):
    wait_deg()
    return carry

  lax.fori_loop(0, NCHUNK, drain_deg, 0)

  @pl.when(w < NEXTRA)
  def _drain_tail():
    wait_deg()

  plsc.subcore_barrier()

  pltpu.sync_copy(agg_sp.at[pl.ds(row_base, ROWS_PER_TILE)],
                  agg_out.at[c, pl.ds(row_base, ROWS_PER_TILE)])
  pltpu.sync_copy(deg_sp.at[pl.ds(row_base, ROWS_PER_TILE)],
                  deg_out.at[c, pl.ds(row_base, ROWS_PER_TILE)])


_sc_agg = pl.kernel(
    _sc_agg_body,
    out_type=(jax.ShapeDtypeStruct((NC, N_SP, OUT_DIM), jnp.float32),
              jax.ShapeDtypeStruct((NC, N_SP, DEG_W), jnp.float32)),
    mesh=_sc_mesh,
    scratch_types=[
        pltpu.VMEM((NCHUNK + 1, CH), jnp.int32),
        pltpu.VMEM((NCHUNK + 1, CH), jnp.int32),
    ] + [
        pltpu.VMEM((CH, OUT_DIM), jnp.float32) for _ in range(6)
    ] + [
        pltpu.VMEM((CH, DEG_W), jnp.float32),
        pltpu.VMEM_SHARED((N_SP, OUT_DIM), jnp.float32),
        pltpu.VMEM_SHARED((N_SP, DEG_W), jnp.float32),
    ] + [pltpu.SemaphoreType.DMA] * 13,
    compiler_params=pltpu.CompilerParams(use_tc_tiling_on_sc=False),
)


def _mm_body(x_ref, wt_ref, o_ref):
  o_ref[...] = jnp.dot(x_ref[...], wt_ref[...],
                       preferred_element_type=jnp.float32)


_MM_BLK = 2000


def _matmul(x, wt):
  return pl.pallas_call(
      _mm_body,
      grid=(N // _MM_BLK,),
      in_specs=[
          pl.BlockSpec((_MM_BLK, IN_DIM), lambda i: (i, 0)),
          pl.BlockSpec((IN_DIM, OUT_DIM), lambda i: (0, 0)),
      ],
      out_specs=pl.BlockSpec((_MM_BLK, OUT_DIM), lambda i: (i, 0)),
      out_shape=jax.ShapeDtypeStruct((N, OUT_DIM), jnp.float32),
  )(x, wt)


FIN_BLK = 80


def _sc_fin_body(agg_hbm, deg_hbm, b_hbm, out_hbm,
                 fa0, fb0, fd0, fe0, fa1, fb1, fd1, fe1, fo, bv,
                 sl0, sl1, so):
  c = lax.axis_index("c")
  s = lax.axis_index("s")
  w = c * NS + s
  rb = w * (FIN_BLK * 4)
  nb = jnp.where(w == NC * NS - 1, 1, 4)

  pltpu.sync_copy(b_hbm, bv)
  bvs = [bv[pl.ds(16 * cc, 16)] for cc in range(OUT_DIM // 16)]
  onef = jnp.full((16,), 1.0, jnp.float32)

  ld = ((fa0, fb0, fd0, fe0, sl0), (fa1, fb1, fd1, fe1, sl1))

  def start_load(i, bufset):
    fa, fb, fd, fe, sl = bufset
    off = rb + i * FIN_BLK
    pltpu.async_copy(agg_hbm.at[0, pl.ds(off, FIN_BLK)], fa, sl)
    pltpu.async_copy(agg_hbm.at[1, pl.ds(off, FIN_BLK)], fb, sl)
    pltpu.async_copy(deg_hbm.at[0, pl.ds(off, FIN_BLK)], fd, sl)
    pltpu.async_copy(deg_hbm.at[1, pl.ds(off, FIN_BLK)], fe, sl)

  def wait_load(bufset):
    fa, fb, fd, fe, sl = bufset
    pltpu.make_async_copy(agg_hbm.at[0, pl.ds(0, FIN_BLK)], fa, sl).wait()
    pltpu.make_async_copy(agg_hbm.at[0, pl.ds(0, FIN_BLK)], fb, sl).wait()
    pltpu.make_async_copy(deg_hbm.at[0, pl.ds(0, FIN_BLK)], fd, sl).wait()
    pltpu.make_async_copy(deg_hbm.at[0, pl.ds(0, FIN_BLK)], fe, sl).wait()

  start_load(0, ld[0])

  def blk(i, carry):
    off = rb + i * FIN_BLK

    @pl.when(i + 1 < nb)
    def _pre():
      for p in range(2):
        @pl.when(lax.rem(i + 1, 2) == p)
        def _():
          start_load(i + 1, ld[p])

    for p in range(2):
      @pl.when(lax.rem(i, 2) == p)
      def _():
        fa, fb, fd, fe, _sl = ld[p]
        wait_load(ld[p])

        def row(r, carry2):
          for u in range(2):
            dv = fd[r + u, pl.ds(0, 16)] + fe[r + u, pl.ds(0, 16)]
            rec = onef / jnp.maximum(dv, onef)
            for cc in range(OUT_DIM // 16):
              a = (fa[r + u, pl.ds(16 * cc, 16)] +
                   fb[r + u, pl.ds(16 * cc, 16)])
              fo[r + u, pl.ds(16 * cc, 16)] = jnp.maximum(
                  a * rec + bvs[cc], 0.0)
          return carry2

        lax.fori_loop(0, FIN_BLK // 2, lambda t, cc2: row(t * 2, cc2), 0)

    @pl.when(i > 0)
    def _wo():
      pltpu.make_async_copy(fo, out_hbm.at[pl.ds(0, FIN_BLK)], so).wait()

    pltpu.async_copy(fo, out_hbm.at[pl.ds(off, FIN_BLK)], so)
    return carry

  lax.fori_loop(0, nb, blk, 0)
  pltpu.make_async_copy(fo, out_hbm.at[pl.ds(0, FIN_BLK)], so).wait()


_sc_fin = pl.kernel(
    _sc_fin_body,
    out_type=jax.ShapeDtypeStruct((N, OUT_DIM), jnp.float32),
    mesh=_sc_mesh,
    scratch_types=[
        pltpu.VMEM((FIN_BLK, OUT_DIM), jnp.float32),
        pltpu.VMEM((FIN_BLK, OUT_DIM), jnp.float32),
        pltpu.VMEM((FIN_BLK, DEG_W), jnp.float32),
        pltpu.VMEM((FIN_BLK, DEG_W), jnp.float32),
        pltpu.VMEM((FIN_BLK, OUT_DIM), jnp.float32),
        pltpu.VMEM((FIN_BLK, OUT_DIM), jnp.float32),
        pltpu.VMEM((FIN_BLK, DEG_W), jnp.float32),
        pltpu.VMEM((FIN_BLK, DEG_W), jnp.float32),
        pltpu.VMEM((FIN_BLK, OUT_DIM), jnp.float32),
        pltpu.VMEM((OUT_DIM,), jnp.float32),
        pltpu.SemaphoreType.DMA,
        pltpu.SemaphoreType.DMA,
        pltpu.SemaphoreType.DMA,
    ],
    compiler_params=pltpu.CompilerParams(use_tc_tiling_on_sc=False),
)


def kernel(node_features, edge_index, W, b):
  ei = edge_index.astype(jnp.int32).reshape(2, NCHUNKS_TOT, CH)
  y = _matmul(node_features, W.T)
  agg2, deg2 = _sc_agg(ei, y)
  return _sc_fin(agg2, deg2, b)
